# Initial kernel scaffold; baseline (speedup 1.0000x reference)
import jax
import jax.numpy as jnp
from jax.experimental import pallas as pl


def kernel(x, edge_index, A0, B0, A1, B1, A2, B2, W_out, b_out):
    raise NotImplementedError("write your pallas kernel here")


# speedup vs baseline: 12.3297x; 12.3297x over previous
import functools

import jax
import jax.numpy as jnp
from jax import lax
from jax.experimental import pallas as pl
from jax.experimental.pallas import tpu as pltpu
from jax.experimental.pallas import tpu_sc as plsc

N = 100000
NP = 102400
E = 1600000
HALF = 16
FW1 = 8
NTILES = 16
ROWS_PER_TILE = NP // NTILES
ZCH = 128
NZ = ROWS_PER_TILE // ZCH
BLK = 8

_mesh = plsc.VectorSubcoreMesh(core_axis_name="c", subcore_axis_name="s")


def _seg_sum_wide(table2n, src2, dst_r, zeros_hbm):
    CH = 100
    EB = CH * BLK
    TILE_BLKS = E // EB // NTILES

    @functools.partial(
        pl.kernel,
        mesh=_mesh,
        compiler_params=pltpu.CompilerParams(use_tc_tiling_on_sc=False),
        out_type=jax.ShapeDtypeStruct((2, NP, HALF), jnp.float32),
        scratch_types=[
            pltpu.VMEM((BLK, CH), jnp.int32),
            pltpu.VMEM((BLK, CH), jnp.int32),
            pltpu.VMEM((EB, HALF), jnp.float32),
            pltpu.VMEM((ZCH, HALF), jnp.float32),
            pltpu.VMEM_SHARED((NP, HALF), jnp.float32),
            pltpu.SemaphoreType.DMA,
        ],
    )
    def k(table_hbm, src_hbm, dst_hbm, z_hbm, out_hbm, src_v, dst_v, rows_v,
          stg_v, acc, sem):
        c = lax.axis_index("c")
        s = lax.axis_index("s")

        pltpu.sync_copy(z_hbm, stg_v)

        def zbody(z, carry):
            r0 = s * ROWS_PER_TILE + z * ZCH
            pltpu.sync_copy(stg_v, acc.at[pl.ds(r0, ZCH)])
            return carry
        lax.fori_loop(0, NZ, zbody, 0)
        plsc.subcore_barrier()

        def ebody(b, carry):
            blk = s * TILE_BLKS + b
            pltpu.sync_copy(src_hbm.at[c, blk], src_v)
            pltpu.sync_copy(dst_hbm.at[blk], dst_v)
            cps = []
            for j in range(BLK):
                cps.append(pltpu.async_copy(
                    table_hbm.at[src_v.at[j]],
                    rows_v.at[pl.ds(j * CH, CH)], sem))
            for cp in cps:
                cp.wait()
            for j in range(BLK):
                pltpu.sync_copy(rows_v.at[pl.ds(j * CH, CH)],
                                acc.at[dst_v.at[j]], add=True)
            return carry
        lax.fori_loop(0, TILE_BLKS, ebody, 0)
        plsc.subcore_barrier()

        def wbody(z, carry):
            r0 = s * ROWS_PER_TILE + z * ZCH
            pltpu.sync_copy(acc.at[pl.ds(r0, ZCH)], stg_v)
            pltpu.sync_copy(stg_v, out_hbm.at[c, pl.ds(r0, ZCH)])
            return carry
        lax.fori_loop(0, NZ, wbody, 0)

    return k(table2n, src2, dst_r, zeros_hbm)


def _seg_sum_in(x_pad, src_r, dst_r, zeros_hbm):
    CH = 125
    EB = CH * BLK
    TILE_BLKS = E // EB // (2 * NTILES)

    @functools.partial(
        pl.kernel,
        mesh=_mesh,
        compiler_params=pltpu.CompilerParams(use_tc_tiling_on_sc=False),
        out_type=jax.ShapeDtypeStruct((2, NP, FW1), jnp.float32),
        scratch_types=[
            pltpu.VMEM((BLK, CH), jnp.int32),
            pltpu.VMEM((BLK, CH), jnp.int32),
            pltpu.VMEM((EB, FW1), jnp.float32),
            pltpu.VMEM((ZCH, FW1), jnp.float32),
            pltpu.VMEM_SHARED((NP, FW1), jnp.float32),
            pltpu.SemaphoreType.DMA,
        ],
    )
    def k(table_hbm, src_hbm, dst_hbm, z_hbm, out_hbm, src_v, dst_v, rows_v,
          stg_v, acc, sem):
        c = lax.axis_index("c")
        s = lax.axis_index("s")

        pltpu.sync_copy(z_hbm, stg_v)

        def zbody(z, carry):
            r0 = s * ROWS_PER_TILE + z * ZCH
            pltpu.sync_copy(stg_v, acc.at[pl.ds(r0, ZCH)])
            return carry
        lax.fori_loop(0, NZ, zbody, 0)
        plsc.subcore_barrier()

        def ebody(b, carry):
            blk = c * (E // EB // 2) + s * TILE_BLKS + b
            pltpu.sync_copy(src_hbm.at[blk], src_v)
            pltpu.sync_copy(dst_hbm.at[blk], dst_v)
            cps = []
            for j in range(BLK):
                cps.append(pltpu.async_copy(
                    table_hbm.at[src_v.at[j]],
                    rows_v.at[pl.ds(j * CH, CH)], sem))
            for cp in cps:
                cp.wait()
            for j in range(BLK):
                pltpu.sync_copy(rows_v.at[pl.ds(j * CH, CH)],
                                acc.at[dst_v.at[j]], add=True)
            return carry
        lax.fori_loop(0, TILE_BLKS, ebody, 0)
        plsc.subcore_barrier()

        def wbody(z, carry):
            r0 = s * ROWS_PER_TILE + z * ZCH
            pltpu.sync_copy(acc.at[pl.ds(r0, ZCH)], stg_v)
            pltpu.sync_copy(stg_v, out_hbm.at[c, pl.ds(r0, ZCH)])
            return carry
        lax.fori_loop(0, NZ, wbody, 0)

    return k(x_pad, src_r, dst_r, zeros_hbm)


R = 5120


def _tc_in(S1p, x, A0, B0):
    def body(sp_ref, x_ref, a_ref, b_ref, h_ref, inv_ref):
        Ssum = sp_ref[0] + sp_ref[1]
        cnt = Ssum[:, 4:5]
        inv = 1.0 / jnp.maximum(cnt, 1.0)
        agg = Ssum[:, 0:4] * inv
        h = (jnp.dot(agg, a_ref[...], preferred_element_type=jnp.float32)
             + jnp.dot(x_ref[...], b_ref[...],
                       preferred_element_type=jnp.float32))
        h_ref[0] = h[:, :HALF]
        h_ref[1] = h[:, HALF:]
        inv_ref[...] = inv

    return pl.pallas_call(
        body,
        grid=(NP // R,),
        in_specs=[
            pl.BlockSpec((2, R, FW1), lambda i: (0, i, 0)),
            pl.BlockSpec((R, 4), lambda i: (i, 0)),
            pl.BlockSpec((4, 32), lambda i: (0, 0)),
            pl.BlockSpec((4, 32), lambda i: (0, 0)),
        ],
        out_specs=[
            pl.BlockSpec((2, R, HALF), lambda i: (0, i, 0)),
            pl.BlockSpec((R, 1), lambda i: (i, 0)),
        ],
        out_shape=[
            jax.ShapeDtypeStruct((2, NP, HALF), jnp.float32),
            jax.ShapeDtypeStruct((NP, 1), jnp.float32),
        ],
    )(S1p, x, A0, B0)


def _tc_mid(S, h, inv, A, B):
    def body(s_ref, h_ref, inv_ref, a_ref, b_ref, o_ref):
        Sfull = jnp.concatenate([s_ref[0], s_ref[1]], axis=1)
        agg = Sfull * inv_ref[...]
        hh = jnp.concatenate([h_ref[0], h_ref[1]], axis=1)
        o = jax.nn.sigmoid(
            jnp.dot(agg, a_ref[...], preferred_element_type=jnp.float32)
            + jnp.dot(hh, b_ref[...], preferred_element_type=jnp.float32))
        o_ref[0] = o[:, :HALF]
        o_ref[1] = o[:, HALF:]

    return pl.pallas_call(
        body,
        grid=(NP // R,),
        in_specs=[
            pl.BlockSpec((2, R, HALF), lambda i: (0, i, 0)),
            pl.BlockSpec((2, R, HALF), lambda i: (0, i, 0)),
            pl.BlockSpec((R, 1), lambda i: (i, 0)),
            pl.BlockSpec((32, 32), lambda i: (0, 0)),
            pl.BlockSpec((32, 32), lambda i: (0, 0)),
        ],
        out_specs=pl.BlockSpec((2, R, HALF), lambda i: (0, i, 0)),
        out_shape=jax.ShapeDtypeStruct((2, NP, HALF), jnp.float32),
    )(S, h, inv, A, B)


def _tc_out(S, h, inv, A, B, W, bias):
    def body(s_ref, h_ref, inv_ref, a_ref, b_ref, w_ref, bias_ref, o_ref):
        Sfull = jnp.concatenate([s_ref[0], s_ref[1]], axis=1)
        agg = Sfull * inv_ref[...]
        hh = jnp.concatenate([h_ref[0], h_ref[1]], axis=1)
        o = jax.nn.sigmoid(
            jnp.dot(agg, a_ref[...], preferred_element_type=jnp.float32)
            + jnp.dot(hh, b_ref[...], preferred_element_type=jnp.float32))
        o_ref[...] = (jnp.dot(o, w_ref[...],
                              preferred_element_type=jnp.float32)
                      + bias_ref[0, 0])

    return pl.pallas_call(
        body,
        grid=(NP // R,),
        in_specs=[
            pl.BlockSpec((2, R, HALF), lambda i: (0, i, 0)),
            pl.BlockSpec((2, R, HALF), lambda i: (0, i, 0)),
            pl.BlockSpec((R, 1), lambda i: (i, 0)),
            pl.BlockSpec((32, 32), lambda i: (0, 0)),
            pl.BlockSpec((32, 32), lambda i: (0, 0)),
            pl.BlockSpec((32, 1), lambda i: (0, 0)),
            pl.BlockSpec((1, 1), lambda i: (0, 0)),
        ],
        out_specs=pl.BlockSpec((R, 1), lambda i: (i, 0)),
        out_shape=jax.ShapeDtypeStruct((NP, 1), jnp.float32),
    )(S, h, inv, A, B, W, bias)


def kernel(x, edge_index, A0, B0, A1, B1, A2, B2, W_out, b_out):
    src = edge_index[0].astype(jnp.int32)
    dst = edge_index[1].astype(jnp.int32)
    src_in = src.reshape(E // (8 * 125), 8, 125)
    dst_in = dst.reshape(E // (8 * 125), 8, 125)
    src2 = jnp.stack([src, src + NP]).reshape(2, E // 800, 8, 100)
    dst_w = dst.reshape(E // 800, 8, 100)
    x_pad = jnp.concatenate(
        [x, jnp.ones((N, 1), jnp.float32), jnp.zeros((N, 3), jnp.float32)],
        axis=1)
    xp = jnp.pad(x, ((0, NP - N), (0, 0)))
    z16 = jnp.zeros((ZCH, HALF), jnp.float32)
    z8 = jnp.zeros((ZCH, FW1), jnp.float32)

    S1p = _seg_sum_in(x_pad, src_in, dst_in, z8)
    h0, inv = _tc_in(S1p, xp, A0, B0)
    S2 = _seg_sum_wide(h0.reshape(2 * NP, HALF), src2, dst_w, z16)
    h1 = _tc_mid(S2, h0, inv, A1, B1)
    S3 = _seg_sum_wide(h1.reshape(2 * NP, HALF), src2, dst_w, z16)
    out = _tc_out(S3, h1, inv, A2, B2, W_out, b_out.reshape(1, 1))
    return out[:N, 0]


# speedup vs baseline: 15.3984x; 1.2489x over previous
import functools

import jax
import jax.numpy as jnp
from jax import lax
from jax.experimental import pallas as pl
from jax.experimental.pallas import tpu as pltpu
from jax.experimental.pallas import tpu_sc as plsc

N = 100000
NP = 102400
E = 1600000
E2 = 1638400
PAGES = E2 // 1024
HALF = 16
FW1 = 8
NTILES = 16
ROWS_PER_TILE = NP // NTILES
CH = 128
BLK = 8
EB = CH * BLK
HB = BLK // 2
EBH = CH * HB

_mesh = plsc.VectorSubcoreMesh(core_axis_name="c", subcore_axis_name="s")
_sc_params = pltpu.CompilerParams(use_tc_tiling_on_sc=False)


def _edge_loop(tab_hbm, src_hbm, dst_hbm, src_v, dst_v, rows_v, acc, sem,
               p0, ptile):
    def copy_idx(page, buf):
        pltpu.sync_copy(src_hbm.at[page], src_v.at[buf])
        pltpu.sync_copy(dst_hbm.at[page], dst_v.at[buf])

    def fire(ibuf, half, rbuf):
        for j in range(HB):
            pltpu.async_copy(tab_hbm.at[src_v.at[ibuf, half * HB + j]],
                             rows_v.at[rbuf, pl.ds(j * CH, CH)], sem)

    def drain(rbuf):
        pltpu.make_async_copy(tab_hbm.at[pl.ds(0, EBH)],
                              rows_v.at[rbuf], sem).wait()

    def scatter(ibuf, half, rbuf):
        for j in range(HB):
            pltpu.sync_copy(rows_v.at[rbuf, pl.ds(j * CH, CH)],
                            acc.at[dst_v.at[ibuf, half * HB + j]], add=True)

    copy_idx(p0, 0)
    fire(0, 0, 0)

    @pl.loop(0, ptile, step=2)
    def _(p):
        for b in range(2):
            pg = p + b

            @pl.when(pg + 1 < ptile)
            def _():
                copy_idx(p0 + pg + 1, 1 - b)

            for h in range(2):
                drain(h)
                if h == 0:
                    fire(b, 1, 1)
                else:
                    @pl.when(pg + 1 < ptile)
                    def _():
                        fire(1 - b, 0, 0)
                scatter(b, h, h)


def _seg_sum_wide(tabL, tabR, src_pg, dst_pg, zeros_hbm):
    PTILE = PAGES // NTILES

    @functools.partial(
        pl.kernel,
        mesh=_mesh,
        compiler_params=_sc_params,
        out_type=jax.ShapeDtypeStruct((2, NP, HALF), jnp.float32),
        scratch_types=[
            pltpu.VMEM((2, BLK, CH), jnp.int32),
            pltpu.VMEM((2, BLK, CH), jnp.int32),
            pltpu.VMEM((2, EBH, HALF), jnp.float32),
            pltpu.VMEM((CH, HALF), jnp.float32),
            pltpu.VMEM_SHARED((NP, HALF), jnp.float32),
            pltpu.SemaphoreType.DMA,
        ],
    )
    def k(tabL_hbm, tabR_hbm, src_hbm, dst_hbm, z_hbm, out_hbm, src_v, dst_v,
          rows_v, stg_v, acc, sem):
        c = lax.axis_index("c")
        s = lax.axis_index("s")
        r0 = s * ROWS_PER_TILE

        pltpu.sync_copy(z_hbm, stg_v)

        @pl.loop(0, ROWS_PER_TILE // CH)
        def _(z):
            pltpu.sync_copy(stg_v, acc.at[pl.ds(r0 + z * CH, CH)])
        plsc.subcore_barrier()

        p0 = s * PTILE

        @pl.when(c == 0)
        def _():
            _edge_loop(tabL_hbm, src_hbm, dst_hbm, src_v, dst_v, rows_v,
                       acc, sem, p0, PTILE)

        @pl.when(c == 1)
        def _():
            _edge_loop(tabR_hbm, src_hbm, dst_hbm, src_v, dst_v, rows_v,
                       acc, sem, p0, PTILE)
        plsc.subcore_barrier()

        @pl.loop(0, ROWS_PER_TILE // CH)
        def _(z):
            pltpu.sync_copy(acc.at[pl.ds(r0 + z * CH, CH)], stg_v)
            pltpu.sync_copy(stg_v, out_hbm.at[c, pl.ds(r0 + z * CH, CH)])

    return k(tabL, tabR, src_pg, dst_pg, zeros_hbm)


def _seg_sum_in(x_pad, src_pg, dst_pg, zeros_hbm):
    PTILE = PAGES // (2 * NTILES)

    @functools.partial(
        pl.kernel,
        mesh=_mesh,
        compiler_params=_sc_params,
        out_type=jax.ShapeDtypeStruct((2, NP, FW1), jnp.float32),
        scratch_types=[
            pltpu.VMEM((2, BLK, CH), jnp.int32),
            pltpu.VMEM((2, BLK, CH), jnp.int32),
            pltpu.VMEM((2, EBH, FW1), jnp.float32),
            pltpu.VMEM((CH, FW1), jnp.float32),
            pltpu.VMEM_SHARED((NP, FW1), jnp.float32),
            pltpu.SemaphoreType.DMA,
        ],
    )
    def k(tab_hbm, src_hbm, dst_hbm, z_hbm, out_hbm, src_v, dst_v, rows_v,
          stg_v, acc, sem):
        c = lax.axis_index("c")
        s = lax.axis_index("s")
        r0 = s * ROWS_PER_TILE

        pltpu.sync_copy(z_hbm, stg_v)

        @pl.loop(0, ROWS_PER_TILE // CH)
        def _(z):
            pltpu.sync_copy(stg_v, acc.at[pl.ds(r0 + z * CH, CH)])
        plsc.subcore_barrier()

        p0 = c * (PAGES // 2) + s * PTILE
        _edge_loop(tab_hbm, src_hbm, dst_hbm, src_v, dst_v, rows_v,
                   acc, sem, p0, PTILE)
        plsc.subcore_barrier()

        @pl.loop(0, ROWS_PER_TILE // CH)
        def _(z):
            pltpu.sync_copy(acc.at[pl.ds(r0 + z * CH, CH)], stg_v)
            pltpu.sync_copy(stg_v, out_hbm.at[c, pl.ds(r0 + z * CH, CH)])

    return k(x_pad, src_pg, dst_pg, zeros_hbm)


R = 5120


def _tc_in(S1p, x, A0, B0):
    def body(sp_ref, x_ref, a_ref, b_ref, hl_ref, hr_ref, inv_ref):
        Ssum = sp_ref[0] + sp_ref[1]
        cnt = Ssum[:, 4:5]
        inv = 1.0 / jnp.maximum(cnt, 1.0)
        agg = Ssum[:, 0:4] * inv
        h = (jnp.dot(agg, a_ref[...], preferred_element_type=jnp.float32)
             + jnp.dot(x_ref[...], b_ref[...],
                       preferred_element_type=jnp.float32))
        hl_ref[...] = h[:, :HALF]
        hr_ref[...] = h[:, HALF:]
        inv_ref[...] = inv

    return pl.pallas_call(
        body,
        grid=(NP // R,),
        in_specs=[
            pl.BlockSpec((2, R, FW1), lambda i: (0, i, 0)),
            pl.BlockSpec((R, 4), lambda i: (i, 0)),
            pl.BlockSpec((4, 32), lambda i: (0, 0)),
            pl.BlockSpec((4, 32), lambda i: (0, 0)),
        ],
        out_specs=[
            pl.BlockSpec((R, HALF), lambda i: (i, 0)),
            pl.BlockSpec((R, HALF), lambda i: (i, 0)),
            pl.BlockSpec((R, 1), lambda i: (i, 0)),
        ],
        out_shape=[
            jax.ShapeDtypeStruct((NP, HALF), jnp.float32),
            jax.ShapeDtypeStruct((NP, HALF), jnp.float32),
            jax.ShapeDtypeStruct((NP, 1), jnp.float32),
        ],
    )(S1p, x, A0, B0)


def _tc_mid(S, hl, hr, inv, A, B):
    def body(s_ref, hl_ref, hr_ref, inv_ref, a_ref, b_ref, ol_ref, or_ref):
        Sfull = jnp.concatenate([s_ref[0], s_ref[1]], axis=1)
        agg = Sfull * inv_ref[...]
        hh = jnp.concatenate([hl_ref[...], hr_ref[...]], axis=1)
        o = jax.nn.sigmoid(
            jnp.dot(agg, a_ref[...], preferred_element_type=jnp.float32)
            + jnp.dot(hh, b_ref[...], preferred_element_type=jnp.float32))
        ol_ref[...] = o[:, :HALF]
        or_ref[...] = o[:, HALF:]

    return pl.pallas_call(
        body,
        grid=(NP // R,),
        in_specs=[
            pl.BlockSpec((2, R, HALF), lambda i: (0, i, 0)),
            pl.BlockSpec((R, HALF), lambda i: (i, 0)),
            pl.BlockSpec((R, HALF), lambda i: (i, 0)),
            pl.BlockSpec((R, 1), lambda i: (i, 0)),
            pl.BlockSpec((32, 32), lambda i: (0, 0)),
            pl.BlockSpec((32, 32), lambda i: (0, 0)),
        ],
        out_specs=[
            pl.BlockSpec((R, HALF), lambda i: (i, 0)),
            pl.BlockSpec((R, HALF), lambda i: (i, 0)),
        ],
        out_shape=[
            jax.ShapeDtypeStruct((NP, HALF), jnp.float32),
            jax.ShapeDtypeStruct((NP, HALF), jnp.float32),
        ],
    )(S, hl, hr, inv, A, B)


def _tc_out(S, hl, hr, inv, A, B, W, bias):
    def body(s_ref, hl_ref, hr_ref, inv_ref, a_ref, b_ref, w_ref, bias_ref,
             o_ref):
        Sfull = jnp.concatenate([s_ref[0], s_ref[1]], axis=1)
        agg = Sfull * inv_ref[...]
        hh = jnp.concatenate([hl_ref[...], hr_ref[...]], axis=1)
        o = jax.nn.sigmoid(
            jnp.dot(agg, a_ref[...], preferred_element_type=jnp.float32)
            + jnp.dot(hh, b_ref[...], preferred_element_type=jnp.float32))
        o_ref[...] = (jnp.dot(o, w_ref[...],
                              preferred_element_type=jnp.float32)
                      + bias_ref[0, 0])

    return pl.pallas_call(
        body,
        grid=(NP // R,),
        in_specs=[
            pl.BlockSpec((2, R, HALF), lambda i: (0, i, 0)),
            pl.BlockSpec((R, HALF), lambda i: (i, 0)),
            pl.BlockSpec((R, HALF), lambda i: (i, 0)),
            pl.BlockSpec((R, 1), lambda i: (i, 0)),
            pl.BlockSpec((32, 32), lambda i: (0, 0)),
            pl.BlockSpec((32, 32), lambda i: (0, 0)),
            pl.BlockSpec((32, 1), lambda i: (0, 0)),
            pl.BlockSpec((1, 1), lambda i: (0, 0)),
        ],
        out_specs=pl.BlockSpec((R, 1), lambda i: (i, 0)),
        out_shape=jax.ShapeDtypeStruct((NP, 1), jnp.float32),
    )(S, hl, hr, inv, A, B, W, bias)


def kernel(x, edge_index, A0, B0, A1, B1, A2, B2, W_out, b_out):
    src = edge_index[0].astype(jnp.int32)
    dst = edge_index[1].astype(jnp.int32)
    npad = E2 - E
    pad_src = (jnp.arange(npad, dtype=jnp.int32) * 13) % N
    pad_dst = N + (jnp.arange(npad, dtype=jnp.int32) % (NP - N))
    src_pg = jnp.concatenate([src, pad_src]).reshape(PAGES, BLK, CH)
    dst_pg = jnp.concatenate([dst, pad_dst]).reshape(PAGES, BLK, CH)
    x_pad = jnp.concatenate(
        [x, jnp.ones((N, 1), jnp.float32), jnp.zeros((N, 3), jnp.float32)],
        axis=1)
    xp = jnp.pad(x, ((0, NP - N), (0, 0)))
    z16 = jnp.zeros((CH, HALF), jnp.float32)
    z8 = jnp.zeros((CH, FW1), jnp.float32)

    S1p = _seg_sum_in(x_pad, src_pg, dst_pg, z8)
    h0l, h0r, inv = _tc_in(S1p, xp, A0, B0)
    S2 = _seg_sum_wide(h0l, h0r, src_pg, dst_pg, z16)
    h1l, h1r = _tc_mid(S2, h0l, h0r, inv, A1, B1)
    S3 = _seg_sum_wide(h1l, h1r, src_pg, dst_pg, z16)
    out = _tc_out(S3, h1l, h1r, inv, A2, B2, W_out, b_out.reshape(1, 1))
    return out[:N, 0]


# speedup vs baseline: 22.8192x; 1.4819x over previous
import functools

import jax
import jax.numpy as jnp
from jax import lax
from jax.experimental import pallas as pl
from jax.experimental.pallas import tpu as pltpu
from jax.experimental.pallas import tpu_sc as plsc

N = 100000
NP = 102400
E = 1600000
E2 = 1638400
PAGES = E2 // 1024
IDXR = E2 // 128
HALF = 16
FW1 = 8
NTILES = 16
ROWS_PER_TILE = NP // NTILES
CH = 128
BLK = 8
EB = CH * BLK
HB = BLK // 2
EBH = CH * HB

_mesh = plsc.VectorSubcoreMesh(core_axis_name="c", subcore_axis_name="s")
_sc_params = pltpu.CompilerParams(use_tc_tiling_on_sc=False)


def _edge_loop(tab_hbm, src_hbm, dst_hbm, src_v, dst_v, rows_v, acc, sem,
               p0, ptile):
    def copy_idx(page, buf):
        pltpu.sync_copy(src_hbm.at[pl.ds(page * 8, 8)], src_v.at[buf])
        pltpu.sync_copy(dst_hbm.at[pl.ds(page * 8, 8)], dst_v.at[buf])

    def fire(ibuf, half, rbuf):
        for j in range(HB):
            pltpu.async_copy(tab_hbm.at[src_v.at[ibuf, half * HB + j]],
                             rows_v.at[rbuf, pl.ds(j * CH, CH)], sem)

    def drain(rbuf):
        pltpu.make_async_copy(tab_hbm.at[pl.ds(0, EBH)],
                              rows_v.at[rbuf], sem).wait()

    def scatter(ibuf, half, rbuf):
        for j in range(HB):
            pltpu.sync_copy(rows_v.at[rbuf, pl.ds(j * CH, CH)],
                            acc.at[dst_v.at[ibuf, half * HB + j]], add=True)

    copy_idx(p0, 0)
    fire(0, 0, 0)

    @pl.loop(0, ptile, step=2)
    def _(p):
        for b in range(2):
            pg = p + b

            @pl.when(pg + 1 < ptile)
            def _():
                copy_idx(p0 + pg + 1, 1 - b)

            for h in range(2):
                drain(h)
                if h == 0:
                    fire(b, 1, 1)
                else:
                    @pl.when(pg + 1 < ptile)
                    def _():
                        fire(1 - b, 0, 0)
                scatter(b, h, h)


def _seg_sum_wide(tabL, tabR, src_pg, dst_pg, zeros_hbm):
    PTILE = PAGES // NTILES

    @functools.partial(
        pl.kernel,
        mesh=_mesh,
        compiler_params=_sc_params,
        out_type=jax.ShapeDtypeStruct((2, NP, HALF), jnp.float32),
        scratch_types=[
            pltpu.VMEM((2, BLK, CH), jnp.int32),
            pltpu.VMEM((2, BLK, CH), jnp.int32),
            pltpu.VMEM((2, EBH, HALF), jnp.float32),
            pltpu.VMEM((CH, HALF), jnp.float32),
            pltpu.VMEM_SHARED((NP, HALF), jnp.float32),
            pltpu.SemaphoreType.DMA,
        ],
    )
    def k(tabL_hbm, tabR_hbm, src_hbm, dst_hbm, z_hbm, out_hbm, src_v, dst_v,
          rows_v, stg_v, acc, sem):
        c = lax.axis_index("c")
        s = lax.axis_index("s")
        r0 = s * ROWS_PER_TILE

        pltpu.sync_copy(z_hbm, stg_v)

        @pl.loop(0, ROWS_PER_TILE // CH)
        def _(z):
            pltpu.sync_copy(stg_v, acc.at[pl.ds(r0 + z * CH, CH)])
        plsc.subcore_barrier()

        p0 = s * PTILE

        @pl.when(c == 0)
        def _():
            _edge_loop(tabL_hbm, src_hbm, dst_hbm, src_v, dst_v, rows_v,
                       acc, sem, p0, PTILE)

        @pl.when(c == 1)
        def _():
            _edge_loop(tabR_hbm, src_hbm, dst_hbm, src_v, dst_v, rows_v,
                       acc, sem, p0, PTILE)
        plsc.subcore_barrier()

        @pl.loop(0, ROWS_PER_TILE // CH)
        def _(z):
            pltpu.sync_copy(acc.at[pl.ds(r0 + z * CH, CH)], stg_v)
            pltpu.sync_copy(stg_v, out_hbm.at[c, pl.ds(r0 + z * CH, CH)])

    return k(tabL, tabR, src_pg, dst_pg, zeros_hbm)


def _seg_sum_in(x_pad, src_pg, dst_pg, zeros_hbm):
    PTILE = PAGES // (2 * NTILES)

    @functools.partial(
        pl.kernel,
        mesh=_mesh,
        compiler_params=_sc_params,
        out_type=jax.ShapeDtypeStruct((2, NP, FW1), jnp.float32),
        scratch_types=[
            pltpu.VMEM((2, BLK, CH), jnp.int32),
            pltpu.VMEM((2, BLK, CH), jnp.int32),
            pltpu.VMEM((2, EBH, FW1), jnp.float32),
            pltpu.VMEM((CH, FW1), jnp.float32),
            pltpu.VMEM_SHARED((NP, FW1), jnp.float32),
            pltpu.SemaphoreType.DMA,
        ],
    )
    def k(tab_hbm, src_hbm, dst_hbm, z_hbm, out_hbm, src_v, dst_v, rows_v,
          stg_v, acc, sem):
        c = lax.axis_index("c")
        s = lax.axis_index("s")
        r0 = s * ROWS_PER_TILE

        pltpu.sync_copy(z_hbm, stg_v)

        @pl.loop(0, ROWS_PER_TILE // CH)
        def _(z):
            pltpu.sync_copy(stg_v, acc.at[pl.ds(r0 + z * CH, CH)])
        plsc.subcore_barrier()

        p0 = c * (PAGES // 2) + s * PTILE
        _edge_loop(tab_hbm, src_hbm, dst_hbm, src_v, dst_v, rows_v,
                   acc, sem, p0, PTILE)
        plsc.subcore_barrier()

        @pl.loop(0, ROWS_PER_TILE // CH)
        def _(z):
            pltpu.sync_copy(acc.at[pl.ds(r0 + z * CH, CH)], stg_v)
            pltpu.sync_copy(stg_v, out_hbm.at[c, pl.ds(r0 + z * CH, CH)])

    return k(x_pad, src_pg, dst_pg, zeros_hbm)


GP = 25
EBLK = E // GP
PADB = (E2 - E) // GP


def _tc_prep(edge_index):
    def body(e_ref, src_ref, dst_ref):
        pid = pl.program_id(0)
        e = e_ref[...]
        srcr = e[0:1, :].reshape(EBLK // 128, 128)
        dstr = e[1:2, :].reshape(EBLK // 128, 128)
        li = (lax.broadcasted_iota(jnp.int32, (PADB // 128, 128), 0) * 128
              + lax.broadcasted_iota(jnp.int32, (PADB // 128, 128), 1)
              + pid * PADB)
        pad_src = (li * 13) % N
        pad_dst = N + li % (NP - N)
        src_ref[...] = jnp.concatenate([srcr, pad_src], axis=0)
        dst_ref[...] = jnp.concatenate([dstr, pad_dst], axis=0)

    RB = (EBLK + PADB) // 128
    return pl.pallas_call(
        body,
        grid=(GP,),
        in_specs=[
            pl.BlockSpec((2, EBLK), lambda i: (0, i)),
        ],
        out_specs=[
            pl.BlockSpec((RB, 128), lambda i: (i, 0)),
            pl.BlockSpec((RB, 128), lambda i: (i, 0)),
        ],
        out_shape=[
            jax.ShapeDtypeStruct((IDXR, 128), jnp.int32),
            jax.ShapeDtypeStruct((IDXR, 128), jnp.int32),
        ],
    )(edge_index)


R = 5120
RS = R // 16
RH = R // 8

import numpy as _np
_Pnp = _np.zeros((8, 16, 128), _np.float32)
for _q in range(8):
    for _k in range(16):
        _Pnp[_q, _k, _q * 16 + _k] = 1.0


def _tc_in(S1p, xp, DA_La, DA_Lb, DA_Ra, DA_Rb, B0L, B0R, Wc, E8, Ea, Eb, P):
    def body(sp_ref, x_ref, dla_ref, dlb_ref, dra_ref, drb_ref, b0l_ref,
             b0r_ref, wc_ref, e8_ref, ea_ref, eb_ref, p_ref,
             hl_ref, hr_ref, inv_ref):
        f32 = jnp.float32
        Sp = sp_ref[0] + sp_ref[1]
        cnt = jnp.dot(Sp, wc_ref[...], preferred_element_type=f32)
        inv16 = 1.0 / jnp.maximum(cnt, 1.0)
        invp8 = jnp.dot(inv16, e8_ref[...], preferred_element_type=f32)
        aggp = Sp * invp8
        xb = x_ref[...]

        def half(da, db, b0h):
            ha = jnp.dot(aggp, da, preferred_element_type=f32)
            hb = jnp.dot(aggp, db, preferred_element_type=f32)
            hp = jnp.stack([ha, hb], axis=1).reshape(RH, 128)
            hx = jnp.dot(xb, b0h, preferred_element_type=f32)
            hxr = hx.reshape(RH, 8, 16)
            for q in range(8):
                hp = hp + jnp.dot(hxr[:, q, :], p_ref[q],
                                  preferred_element_type=f32)
            return hp

        hl_ref[...] = half(dla_ref[...], dlb_ref[...], b0l_ref[...])
        hr_ref[...] = half(dra_ref[...], drb_ref[...], b0r_ref[...])
        ia = jnp.dot(inv16, ea_ref[...], preferred_element_type=f32)
        ib = jnp.dot(inv16, eb_ref[...], preferred_element_type=f32)
        inv_ref[...] = jnp.stack([ia, ib], axis=1).reshape(RH, 128)

    full = lambda shape: pl.BlockSpec(shape, lambda i: tuple(0 for _ in shape))
    return pl.pallas_call(
        body,
        grid=(NP // R,),
        in_specs=[
            pl.BlockSpec((2, RS, 128), lambda i: (0, i, 0)),
            pl.BlockSpec((R, 4), lambda i: (i, 0)),
            full((128, 128)), full((128, 128)), full((128, 128)),
            full((128, 128)), full((4, 16)), full((4, 16)),
            full((128, 16)), full((16, 128)), full((16, 128)),
            full((16, 128)), full((8, 16, 128)),
        ],
        out_specs=[
            pl.BlockSpec((RH, 128), lambda i: (i, 0)),
            pl.BlockSpec((RH, 128), lambda i: (i, 0)),
            pl.BlockSpec((RH, 128), lambda i: (i, 0)),
        ],
        out_shape=[
            jax.ShapeDtypeStruct((NP * HALF // 128, 128), jnp.float32),
            jax.ShapeDtypeStruct((NP * HALF // 128, 128), jnp.float32),
            jax.ShapeDtypeStruct((NP * HALF // 128, 128), jnp.float32),
        ],
    )(S1p, xp, DA_La, DA_Lb, DA_Ra, DA_Rb, B0L, B0R, Wc, E8, Ea, Eb, P)


def _tc_mid(S, hl, hr, invp, DAL, DAR, DBL, DBR, DAL2, DAR2, DBL2, DBR2):
    def body(s_ref, hl_ref, hr_ref, inv_ref, a1_ref, a2_ref, b1_ref, b2_ref,
             a3_ref, a4_ref, b3_ref, b4_ref, ol_ref, or_ref):
        f32 = jnp.float32
        iv = inv_ref[...]
        aggL = s_ref[0] * iv
        aggR = s_ref[1] * iv
        hL = hl_ref[...]
        hR = hr_ref[...]

        def z(da, db, ba, bb):
            return (jnp.dot(aggL, da, preferred_element_type=f32)
                    + jnp.dot(aggR, db, preferred_element_type=f32)
                    + jnp.dot(hL, ba, preferred_element_type=f32)
                    + jnp.dot(hR, bb, preferred_element_type=f32))

        ol_ref[...] = jax.nn.sigmoid(
            z(a1_ref[...], a2_ref[...], b1_ref[...], b2_ref[...]))
        or_ref[...] = jax.nn.sigmoid(
            z(a3_ref[...], a4_ref[...], b3_ref[...], b4_ref[...]))

    full = lambda: pl.BlockSpec((128, 128), lambda i: (0, 0))
    blk = lambda: pl.BlockSpec((RH, 128), lambda i: (i, 0))
    return pl.pallas_call(
        body,
        grid=(NP // R,),
        in_specs=[pl.BlockSpec((2, RH, 128), lambda i: (0, i, 0)),
                  blk(), blk(), blk(),
                  full(), full(), full(), full(),
                  full(), full(), full(), full()],
        out_specs=[blk(), blk()],
        out_shape=[
            jax.ShapeDtypeStruct((NP * HALF // 128, 128), jnp.float32),
            jax.ShapeDtypeStruct((NP * HALF // 128, 128), jnp.float32),
        ],
    )(S, hl, hr, invp, DAL, DAR, DBL, DBR, DAL2, DAR2, DBL2, DBR2)


def _tc_out(S, hl, hr, invp, DAL, DAR, DBL, DBR, DAL2, DAR2, DBL2, DBR2,
            DWl, DWr, bias):
    def body(s_ref, hl_ref, hr_ref, inv_ref, a1_ref, a2_ref, b1_ref, b2_ref,
             a3_ref, a4_ref, b3_ref, b4_ref, wl_ref, wr_ref, bias_ref,
             o_ref):
        f32 = jnp.float32
        iv = inv_ref[...]
        aggL = s_ref[0] * iv
        aggR = s_ref[1] * iv
        hL = hl_ref[...]
        hR = hr_ref[...]

        def z(da, db, ba, bb):
            return (jnp.dot(aggL, da, preferred_element_type=f32)
                    + jnp.dot(aggR, db, preferred_element_type=f32)
                    + jnp.dot(hL, ba, preferred_element_type=f32)
                    + jnp.dot(hR, bb, preferred_element_type=f32))

        oL = jax.nn.sigmoid(
            z(a1_ref[...], a2_ref[...], b1_ref[...], b2_ref[...]))
        oR = jax.nn.sigmoid(
            z(a3_ref[...], a4_ref[...], b3_ref[...], b4_ref[...]))
        dn = (((0,), (1,)), ((), ()))
        tL = lax.dot_general(wl_ref[...], oL, dn, preferred_element_type=f32)
        tR = lax.dot_general(wr_ref[...], oR, dn, preferred_element_type=f32)
        o_ref[...] = tL + tR + bias_ref[0, 0]

    full = lambda: pl.BlockSpec((128, 128), lambda i: (0, 0))
    blk = lambda: pl.BlockSpec((RH, 128), lambda i: (i, 0))
    return pl.pallas_call(
        body,
        grid=(NP // R,),
        in_specs=[pl.BlockSpec((2, RH, 128), lambda i: (0, i, 0)),
                  blk(), blk(), blk(),
                  full(), full(), full(), full(),
                  full(), full(), full(), full(),
                  pl.BlockSpec((128, 8), lambda i: (0, 0)),
                  pl.BlockSpec((128, 8), lambda i: (0, 0)),
                  pl.BlockSpec((1, 1), lambda i: (0, 0))],
        out_specs=pl.BlockSpec((8, RH), lambda i: (0, i)),
        out_shape=jax.ShapeDtypeStruct((8, NP // 8), jnp.float32),
    )(S, hl, hr, invp, DAL, DAR, DBL, DBR, DAL2, DAR2, DBL2, DBR2,
      DWl, DWr, bias)


def kernel(x, edge_index, A0, B0, A1, B1, A2, B2, W_out, b_out):
    f32 = jnp.float32
    ei = edge_index.astype(jnp.int32)
    src_pg, dst_pg = _tc_prep(ei)
    x_pad = jnp.concatenate(
        [x, jnp.ones((N, 1), f32), jnp.zeros((N, 3), f32)],
        axis=1)
    xp = jnp.pad(x, ((0, NP - N), (0, 0)))
    z16 = jnp.zeros((CH, HALF), f32)
    z8 = jnp.zeros((CH, FW1), f32)

    I8 = jnp.eye(8, dtype=f32)
    I16 = jnp.eye(16, dtype=f32)
    A0p = jnp.pad(A0, ((0, 4), (0, 0)))
    M_L = jnp.kron(I16, A0p[:, :HALF])
    M_R = jnp.kron(I16, A0p[:, HALF:])
    E8 = jnp.kron(I16, jnp.ones((1, 8), f32))
    EE = jnp.kron(I16, jnp.ones((1, 16), f32))
    onehot4 = jnp.zeros((8, 1), f32).at[4, 0].set(1.0)
    Wc = jnp.kron(I16, onehot4)
    P = jnp.asarray(_Pnp)

    def dmats(A, B):
        return (jnp.kron(I8, A[:HALF, :HALF]), jnp.kron(I8, A[HALF:, :HALF]),
                jnp.kron(I8, B[:HALF, :HALF]), jnp.kron(I8, B[HALF:, :HALF]),
                jnp.kron(I8, A[:HALF, HALF:]), jnp.kron(I8, A[HALF:, HALF:]),
                jnp.kron(I8, B[:HALF, HALF:]), jnp.kron(I8, B[HALF:, HALF:]))

    DWl = jnp.kron(I8, W_out[:HALF, :])
    DWr = jnp.kron(I8, W_out[HALF:, :])

    S1p = _seg_sum_in(x_pad, src_pg, dst_pg, z8)
    S1v = S1p.reshape(2, NP * FW1 // 128, 128)
    h0l, h0r, invp = _tc_in(S1v, xp, M_L[:, :128], M_L[:, 128:],
                            M_R[:, :128], M_R[:, 128:], B0[:, :HALF],
                            B0[:, HALF:], Wc, E8, EE[:, :128], EE[:, 128:], P)
    S2 = _seg_sum_wide(h0l.reshape(NP, HALF), h0r.reshape(NP, HALF),
                       src_pg, dst_pg, z16)
    S2v = S2.reshape(2, NP * HALF // 128, 128)
    h1l, h1r = _tc_mid(S2v, h0l, h0r, invp, *dmats(A1, B1))
    S3 = _seg_sum_wide(h1l.reshape(NP, HALF), h1r.reshape(NP, HALF),
                       src_pg, dst_pg, z16)
    S3v = S3.reshape(2, NP * HALF // 128, 128)
    out = _tc_out(S3v, h1l, h1r, invp, *dmats(A2, B2), DWl, DWr,
                  b_out.reshape(1, 1))
    return out.T.reshape(NP)[:N]


# speedup vs baseline: 26.4978x; 1.1612x over previous
import functools

import jax
import jax.numpy as jnp
from jax import lax
from jax.experimental import pallas as pl
from jax.experimental.pallas import tpu as pltpu
from jax.experimental.pallas import tpu_sc as plsc

N = 100000
NP = 102400
E = 1600000
E2 = 1638400
PAGES = E2 // 1024
IDXR = E2 // 128
HALF = 16
FW1 = 8
NTILES = 16
ROWS_PER_TILE = NP // NTILES
CH = 128
BLK = 8
EB = CH * BLK
HB = BLK // 2
EBH = CH * HB

_mesh = plsc.VectorSubcoreMesh(core_axis_name="c", subcore_axis_name="s")
_sc_params = pltpu.CompilerParams(use_tc_tiling_on_sc=False)


def _edge_loop(tab_hbm, src_hbm, dst_hbm, src_v, dst_v, rows_v, acc, sem,
               p0, ptile):
    sem_i = sem.at[1]
    sem_g = sem.at[0]

    def copy_idx(page, buf):
        pltpu.async_copy(src_hbm.at[pl.ds(page * 8, 8)], src_v.at[buf],
                         sem_i)
        pltpu.async_copy(dst_hbm.at[pl.ds(page * 8, 8)], dst_v.at[buf],
                         sem_i)

    def wait_idx(buf):
        pltpu.make_async_copy(src_hbm.at[pl.ds(0, 8)], src_v.at[buf],
                              sem_i).wait()
        pltpu.make_async_copy(dst_hbm.at[pl.ds(0, 8)], dst_v.at[buf],
                              sem_i).wait()

    def fire(ibuf, half, rbuf):
        for j in range(HB):
            pltpu.async_copy(tab_hbm.at[src_v.at[ibuf, half * HB + j]],
                             rows_v.at[rbuf, pl.ds(j * CH, CH)], sem_g)

    def drain(rbuf):
        pltpu.make_async_copy(tab_hbm.at[pl.ds(0, EBH)],
                              rows_v.at[rbuf], sem_g).wait()

    def scatter(ibuf, half, rbuf):
        for j in range(HB):
            pltpu.sync_copy(rows_v.at[rbuf, pl.ds(j * CH, CH)],
                            acc.at[dst_v.at[ibuf, half * HB + j]], add=True)

    copy_idx(p0, 0)
    wait_idx(0)
    fire(0, 0, 0)

    @pl.loop(0, ptile, step=2)
    def _(p):
        for b in range(2):
            pg = p + b

            @pl.when(pg + 1 < ptile)
            def _():
                copy_idx(p0 + pg + 1, 1 - b)

            for h in range(2):
                drain(h)
                if h == 0:
                    fire(b, 1, 1)
                else:
                    @pl.when(pg + 1 < ptile)
                    def _():
                        wait_idx(1 - b)
                        fire(1 - b, 0, 0)
                scatter(b, h, h)


def _seg_sum_wide(tabL, tabR, src_pg, dst_pg, zeros_hbm):
    PTILE = PAGES // NTILES

    @functools.partial(
        pl.kernel,
        mesh=_mesh,
        compiler_params=_sc_params,
        out_type=jax.ShapeDtypeStruct((2, NP, HALF), jnp.float32),
        scratch_types=[
            pltpu.VMEM((2, BLK, CH), jnp.int32),
            pltpu.VMEM((2, BLK, CH), jnp.int32),
            pltpu.VMEM((2, EBH, HALF), jnp.float32),
            pltpu.VMEM_SHARED((NP, HALF), jnp.float32),
            pltpu.SemaphoreType.DMA((2,)),
        ],
    )
    def k(tabL_hbm, tabR_hbm, src_hbm, dst_hbm, z_hbm, out_hbm, src_v, dst_v,
          rows_v, acc, sem):
        c = lax.axis_index("c")
        s = lax.axis_index("s")
        r0 = s * ROWS_PER_TILE

        pltpu.sync_copy(z_hbm, acc.at[pl.ds(r0, ROWS_PER_TILE)])
        plsc.subcore_barrier()

        p0 = s * PTILE

        @pl.when(c == 0)
        def _():
            _edge_loop(tabL_hbm, src_hbm, dst_hbm, src_v, dst_v, rows_v,
                       acc, sem, p0, PTILE)

        @pl.when(c == 1)
        def _():
            _edge_loop(tabR_hbm, src_hbm, dst_hbm, src_v, dst_v, rows_v,
                       acc, sem, p0, PTILE)
        plsc.subcore_barrier()

        pltpu.sync_copy(acc.at[pl.ds(r0, ROWS_PER_TILE)],
                        out_hbm.at[c, pl.ds(r0, ROWS_PER_TILE)])

    return k(tabL, tabR, src_pg, dst_pg, zeros_hbm)


def _seg_sum_in(x_pad, src_pg, dst_pg, zeros_hbm):
    PTILE = PAGES // (2 * NTILES)

    @functools.partial(
        pl.kernel,
        mesh=_mesh,
        compiler_params=_sc_params,
        out_type=jax.ShapeDtypeStruct((2, NP, FW1), jnp.float32),
        scratch_types=[
            pltpu.VMEM((2, BLK, CH), jnp.int32),
            pltpu.VMEM((2, BLK, CH), jnp.int32),
            pltpu.VMEM((2, EBH, FW1), jnp.float32),
            pltpu.VMEM_SHARED((NP, FW1), jnp.float32),
            pltpu.SemaphoreType.DMA((2,)),
        ],
    )
    def k(tab_hbm, src_hbm, dst_hbm, z_hbm, out_hbm, src_v, dst_v, rows_v,
          acc, sem):
        c = lax.axis_index("c")
        s = lax.axis_index("s")
        r0 = s * ROWS_PER_TILE

        pltpu.sync_copy(z_hbm, acc.at[pl.ds(r0, ROWS_PER_TILE)])
        plsc.subcore_barrier()

        p0 = c * (PAGES // 2) + s * PTILE
        _edge_loop(tab_hbm, src_hbm, dst_hbm, src_v, dst_v, rows_v,
                   acc, sem, p0, PTILE)
        plsc.subcore_barrier()

        pltpu.sync_copy(acc.at[pl.ds(r0, ROWS_PER_TILE)],
                        out_hbm.at[c, pl.ds(r0, ROWS_PER_TILE)])

    return k(x_pad, src_pg, dst_pg, zeros_hbm)


GP = 25
EBLK = E // GP
PADB = (E2 - E) // GP


def _tc_prep(edge_index):
    def body(e_ref, src_ref, dst_ref):
        pid = pl.program_id(0)
        e = e_ref[...]
        srcr = e[0:1, :].reshape(EBLK // 128, 128)
        dstr = e[1:2, :].reshape(EBLK // 128, 128)
        li = (lax.broadcasted_iota(jnp.int32, (PADB // 128, 128), 0) * 128
              + lax.broadcasted_iota(jnp.int32, (PADB // 128, 128), 1)
              + pid * PADB)
        pad_src = (li * 13) % N
        pad_dst = N + li % (NP - N)
        src_ref[...] = jnp.concatenate([srcr, pad_src], axis=0)
        dst_ref[...] = jnp.concatenate([dstr, pad_dst], axis=0)

    RB = (EBLK + PADB) // 128
    return pl.pallas_call(
        body,
        grid=(GP,),
        in_specs=[
            pl.BlockSpec((2, EBLK), lambda i: (0, i)),
        ],
        out_specs=[
            pl.BlockSpec((RB, 128), lambda i: (i, 0)),
            pl.BlockSpec((RB, 128), lambda i: (i, 0)),
        ],
        out_shape=[
            jax.ShapeDtypeStruct((IDXR, 128), jnp.int32),
            jax.ShapeDtypeStruct((IDXR, 128), jnp.int32),
        ],
    )(edge_index)


R = 5120
RS = R // 16
RH = R // 8

import numpy as _np
_Pnp = _np.zeros((8, 16, 128), _np.float32)
for _q in range(8):
    for _k in range(16):
        _Pnp[_q, _k, _q * 16 + _k] = 1.0


def _tc_in(S1p, xp, DA_La, DA_Lb, DA_Ra, DA_Rb, B0L, B0R, Wc, E8, Ea, Eb, P):
    def body(sp_ref, x_ref, dla_ref, dlb_ref, dra_ref, drb_ref, b0l_ref,
             b0r_ref, wc_ref, e8_ref, ea_ref, eb_ref, p_ref,
             hl_ref, hr_ref, inv_ref):
        f32 = jnp.float32
        Sp = sp_ref[0] + sp_ref[1]
        cnt = jnp.dot(Sp, wc_ref[...], preferred_element_type=f32)
        inv16 = 1.0 / jnp.maximum(cnt, 1.0)
        invp8 = jnp.dot(inv16, e8_ref[...], preferred_element_type=f32)
        aggp = Sp * invp8
        xb = x_ref[...]
        dn = (((0,), (0,)), ((), ()))

        def half(da, db, b0h):
            ha = jnp.dot(aggp, da, preferred_element_type=f32)
            hb = jnp.dot(aggp, db, preferred_element_type=f32)
            hp = jnp.stack([ha, hb], axis=1).reshape(RH, 128)
            hx = lax.dot_general(xb, b0h, dn,
                                 preferred_element_type=f32)
            hxr = hx.reshape(RH, 8, 16)
            for q in range(8):
                hp = hp + jnp.dot(hxr[:, q, :], p_ref[q],
                                  preferred_element_type=f32)
            return hp

        hl_ref[...] = half(dla_ref[...], dlb_ref[...], b0l_ref[...])
        hr_ref[...] = half(dra_ref[...], drb_ref[...], b0r_ref[...])
        ia = jnp.dot(inv16, ea_ref[...], preferred_element_type=f32)
        ib = jnp.dot(inv16, eb_ref[...], preferred_element_type=f32)
        inv_ref[...] = jnp.stack([ia, ib], axis=1).reshape(RH, 128)

    full = lambda shape: pl.BlockSpec(shape, lambda i: tuple(0 for _ in shape))
    return pl.pallas_call(
        body,
        grid=(NP // R,),
        in_specs=[
            pl.BlockSpec((2, RS, 128), lambda i: (0, i, 0)),
            pl.BlockSpec((4, R), lambda i: (0, i)),
            full((128, 128)), full((128, 128)), full((128, 128)),
            full((128, 128)), full((4, 16)), full((4, 16)),
            full((128, 16)), full((16, 128)), full((16, 128)),
            full((16, 128)), full((8, 16, 128)),
        ],
        out_specs=[
            pl.BlockSpec((RH, 128), lambda i: (i, 0)),
            pl.BlockSpec((RH, 128), lambda i: (i, 0)),
            pl.BlockSpec((RH, 128), lambda i: (i, 0)),
        ],
        out_shape=[
            jax.ShapeDtypeStruct((NP * HALF // 128, 128), jnp.float32),
            jax.ShapeDtypeStruct((NP * HALF // 128, 128), jnp.float32),
            jax.ShapeDtypeStruct((NP * HALF // 128, 128), jnp.float32),
        ],
    )(S1p, xp, DA_La, DA_Lb, DA_Ra, DA_Rb, B0L, B0R, Wc, E8, Ea, Eb, P)


def _tc_mid(S, hl, hr, invp, DAL, DAR, DBL, DBR, DAL2, DAR2, DBL2, DBR2):
    def body(s_ref, hl_ref, hr_ref, inv_ref, a1_ref, a2_ref, b1_ref, b2_ref,
             a3_ref, a4_ref, b3_ref, b4_ref, ol_ref, or_ref):
        f32 = jnp.float32
        iv = inv_ref[...]
        aggL = s_ref[0] * iv
        aggR = s_ref[1] * iv
        hL = hl_ref[...]
        hR = hr_ref[...]

        def z(da, db, ba, bb):
            return (jnp.dot(aggL, da, preferred_element_type=f32)
                    + jnp.dot(aggR, db, preferred_element_type=f32)
                    + jnp.dot(hL, ba, preferred_element_type=f32)
                    + jnp.dot(hR, bb, preferred_element_type=f32))

        ol_ref[...] = jax.nn.sigmoid(
            z(a1_ref[...], a2_ref[...], b1_ref[...], b2_ref[...]))
        or_ref[...] = jax.nn.sigmoid(
            z(a3_ref[...], a4_ref[...], b3_ref[...], b4_ref[...]))

    full = lambda: pl.BlockSpec((128, 128), lambda i: (0, 0))
    blk = lambda: pl.BlockSpec((RH, 128), lambda i: (i, 0))
    return pl.pallas_call(
        body,
        grid=(NP // R,),
        in_specs=[pl.BlockSpec((2, RH, 128), lambda i: (0, i, 0)),
                  blk(), blk(), blk(),
                  full(), full(), full(), full(),
                  full(), full(), full(), full()],
        out_specs=[blk(), blk()],
        out_shape=[
            jax.ShapeDtypeStruct((NP * HALF // 128, 128), jnp.float32),
            jax.ShapeDtypeStruct((NP * HALF // 128, 128), jnp.float32),
        ],
    )(S, hl, hr, invp, DAL, DAR, DBL, DBR, DAL2, DAR2, DBL2, DBR2)


def _tc_out(S, hl, hr, invp, DAL, DAR, DBL, DBR, DAL2, DAR2, DBL2, DBR2,
            DWl, DWr, bias):
    def body(s_ref, hl_ref, hr_ref, inv_ref, a1_ref, a2_ref, b1_ref, b2_ref,
             a3_ref, a4_ref, b3_ref, b4_ref, wl_ref, wr_ref, bias_ref,
             o_ref):
        f32 = jnp.float32
        iv = inv_ref[...]
        aggL = s_ref[0] * iv
        aggR = s_ref[1] * iv
        hL = hl_ref[...]
        hR = hr_ref[...]

        def z(da, db, ba, bb):
            return (jnp.dot(aggL, da, preferred_element_type=f32)
                    + jnp.dot(aggR, db, preferred_element_type=f32)
                    + jnp.dot(hL, ba, preferred_element_type=f32)
                    + jnp.dot(hR, bb, preferred_element_type=f32))

        oL = jax.nn.sigmoid(
            z(a1_ref[...], a2_ref[...], b1_ref[...], b2_ref[...]))
        oR = jax.nn.sigmoid(
            z(a3_ref[...], a4_ref[...], b3_ref[...], b4_ref[...]))
        dn = (((0,), (1,)), ((), ()))
        tL = lax.dot_general(wl_ref[...], oL, dn, preferred_element_type=f32)
        tR = lax.dot_general(wr_ref[...], oR, dn, preferred_element_type=f32)
        o_ref[...] = tL + tR + bias_ref[0, 0]

    full = lambda: pl.BlockSpec((128, 128), lambda i: (0, 0))
    blk = lambda: pl.BlockSpec((RH, 128), lambda i: (i, 0))
    return pl.pallas_call(
        body,
        grid=(NP // R,),
        in_specs=[pl.BlockSpec((2, RH, 128), lambda i: (0, i, 0)),
                  blk(), blk(), blk(),
                  full(), full(), full(), full(),
                  full(), full(), full(), full(),
                  pl.BlockSpec((128, 8), lambda i: (0, 0)),
                  pl.BlockSpec((128, 8), lambda i: (0, 0)),
                  pl.BlockSpec((1, 1), lambda i: (0, 0))],
        out_specs=pl.BlockSpec((8, RH), lambda i: (0, i)),
        out_shape=jax.ShapeDtypeStruct((8, NP // 8), jnp.float32),
    )(S, hl, hr, invp, DAL, DAR, DBL, DBR, DAL2, DAR2, DBL2, DBR2,
      DWl, DWr, bias)


def kernel(x, edge_index, A0, B0, A1, B1, A2, B2, W_out, b_out):
    f32 = jnp.float32
    ei = edge_index.astype(jnp.int32)
    src_pg, dst_pg = _tc_prep(ei)
    x_pad = jnp.concatenate(
        [x, jnp.ones((N, 1), f32), jnp.zeros((N, 3), f32)],
        axis=1)
    xpT = jnp.pad(x.T, ((0, 0), (0, NP - N)))
    z16 = jnp.zeros((ROWS_PER_TILE, HALF), f32)
    z8 = jnp.zeros((ROWS_PER_TILE, FW1), f32)

    I8 = jnp.eye(8, dtype=f32)
    I16 = jnp.eye(16, dtype=f32)
    A0p = jnp.pad(A0, ((0, 4), (0, 0)))
    M_L = jnp.kron(I16, A0p[:, :HALF])
    M_R = jnp.kron(I16, A0p[:, HALF:])
    E8 = jnp.kron(I16, jnp.ones((1, 8), f32))
    EE = jnp.kron(I16, jnp.ones((1, 16), f32))
    onehot4 = jnp.zeros((8, 1), f32).at[4, 0].set(1.0)
    Wc = jnp.kron(I16, onehot4)
    P = jnp.asarray(_Pnp)

    def dmats(A, B):
        return (jnp.kron(I8, A[:HALF, :HALF]), jnp.kron(I8, A[HALF:, :HALF]),
                jnp.kron(I8, B[:HALF, :HALF]), jnp.kron(I8, B[HALF:, :HALF]),
                jnp.kron(I8, A[:HALF, HALF:]), jnp.kron(I8, A[HALF:, HALF:]),
                jnp.kron(I8, B[:HALF, HALF:]), jnp.kron(I8, B[HALF:, HALF:]))

    DWl = jnp.kron(I8, W_out[:HALF, :])
    DWr = jnp.kron(I8, W_out[HALF:, :])

    S1p = _seg_sum_in(x_pad, src_pg, dst_pg, z8)
    S1v = S1p.reshape(2, NP * FW1 // 128, 128)
    h0l, h0r, invp = _tc_in(S1v, xpT, M_L[:, :128], M_L[:, 128:],
                            M_R[:, :128], M_R[:, 128:], B0[:, :HALF],
                            B0[:, HALF:], Wc, E8, EE[:, :128], EE[:, 128:], P)
    S2 = _seg_sum_wide(h0l.reshape(NP, HALF), h0r.reshape(NP, HALF),
                       src_pg, dst_pg, z16)
    S2v = S2.reshape(2, NP * HALF // 128, 128)
    h1l, h1r = _tc_mid(S2v, h0l, h0r, invp, *dmats(A1, B1))
    S3 = _seg_sum_wide(h1l.reshape(NP, HALF), h1r.reshape(NP, HALF),
                       src_pg, dst_pg, z16)
    S3v = S3.reshape(2, NP * HALF // 128, 128)
    out = _tc_out(S3v, h1l, h1r, invp, *dmats(A2, B2), DWl, DWr,
                  b_out.reshape(1, 1))
    return out.T.reshape(NP)[:N]


# speedup vs baseline: 26.6828x; 1.0070x over previous
import functools

import jax
import jax.numpy as jnp
from jax import lax
from jax.experimental import pallas as pl
from jax.experimental.pallas import tpu as pltpu
from jax.experimental.pallas import tpu_sc as plsc

N = 100000
NP = 102400
E = 1600000
E2 = 1638400
PAGES = E2 // 1024
IDXR = E2 // 128
HALF = 16
FW1 = 8
NTILES = 16
ROWS_PER_TILE = NP // NTILES
CH = 128
BLK = 8
EB = CH * BLK
HB = BLK // 2
EBH = CH * HB

_mesh = plsc.VectorSubcoreMesh(core_axis_name="c", subcore_axis_name="s")
_sc_params = pltpu.CompilerParams(use_tc_tiling_on_sc=False)


def _edge_loop(tab_hbm, src_hbm, dst_hbm, src_v, dst_v, rows_v, acc, sem,
               p0, ptile):
    sem_g = sem.at[0]
    sem_i = sem.at[1]
    sem_s = sem.at[2]

    def copy_idx(page, buf):
        pltpu.async_copy(src_hbm.at[pl.ds(page * 8, 8)], src_v.at[buf],
                         sem_i)
        pltpu.async_copy(dst_hbm.at[pl.ds(page * 8, 8)], dst_v.at[buf],
                         sem_i)

    def wait_idx(buf):
        pltpu.make_async_copy(src_hbm.at[pl.ds(0, 8)], src_v.at[buf],
                              sem_i).wait()
        pltpu.make_async_copy(dst_hbm.at[pl.ds(0, 8)], dst_v.at[buf],
                              sem_i).wait()

    def fire(ibuf, half, rbuf):
        for j in range(HB):
            pltpu.async_copy(tab_hbm.at[src_v.at[ibuf, half * HB + j]],
                             rows_v.at[rbuf, pl.ds(j * CH, CH)], sem_g)

    def drain(rbuf):
        pltpu.make_async_copy(tab_hbm.at[pl.ds(0, EBH)],
                              rows_v.at[rbuf], sem_g).wait()

    def scatter(ibuf, half, rbuf):
        for j in range(HB):
            pltpu.async_copy(rows_v.at[rbuf, pl.ds(j * CH, CH)],
                             acc.at[dst_v.at[ibuf, half * HB + j]], sem_s,
                             add=True)

    def drain_scat(rbuf):
        pltpu.make_async_copy(rows_v.at[rbuf],
                              acc.at[pl.ds(0, EBH)], sem_s).wait()

    copy_idx(p0, 0)
    wait_idx(0)
    fire(0, 0, 0)

    @pl.loop(0, ptile, step=2)
    def _(p):
        for b in range(2):
            pg = p + b

            @pl.when(pg + 1 < ptile)
            def _():
                copy_idx(p0 + pg + 1, 1 - b)

            for h in range(2):
                drain(h)
                if h == 0:
                    @pl.when(pg > 0)
                    def _():
                        drain_scat(1)
                    fire(b, 1, 1)
                else:
                    @pl.when(pg + 1 < ptile)
                    def _():
                        wait_idx(1 - b)
                        drain_scat(0)
                        fire(1 - b, 0, 0)
                scatter(b, h, h)

    drain_scat(0)
    drain_scat(1)


def _seg_sum_wide(tabL, tabR, src_pg, dst_pg, zeros_hbm):
    PTILE = PAGES // NTILES

    @functools.partial(
        pl.kernel,
        mesh=_mesh,
        compiler_params=_sc_params,
        out_type=jax.ShapeDtypeStruct((2, NP, HALF), jnp.float32),
        scratch_types=[
            pltpu.VMEM((2, BLK, CH), jnp.int32),
            pltpu.VMEM((2, BLK, CH), jnp.int32),
            pltpu.VMEM((2, EBH, HALF), jnp.float32),
            pltpu.VMEM_SHARED((NP, HALF), jnp.float32),
            pltpu.SemaphoreType.DMA((3,)),
        ],
    )
    def k(tabL_hbm, tabR_hbm, src_hbm, dst_hbm, z_hbm, out_hbm, src_v, dst_v,
          rows_v, acc, sem):
        c = lax.axis_index("c")
        s = lax.axis_index("s")
        r0 = s * ROWS_PER_TILE

        pltpu.sync_copy(z_hbm, acc.at[pl.ds(r0, ROWS_PER_TILE)])
        plsc.subcore_barrier()

        p0 = s * PTILE

        @pl.when(c == 0)
        def _():
            _edge_loop(tabL_hbm, src_hbm, dst_hbm, src_v, dst_v, rows_v,
                       acc, sem, p0, PTILE)

        @pl.when(c == 1)
        def _():
            _edge_loop(tabR_hbm, src_hbm, dst_hbm, src_v, dst_v, rows_v,
                       acc, sem, p0, PTILE)
        plsc.subcore_barrier()

        pltpu.sync_copy(acc.at[pl.ds(r0, ROWS_PER_TILE)],
                        out_hbm.at[c, pl.ds(r0, ROWS_PER_TILE)])

    return k(tabL, tabR, src_pg, dst_pg, zeros_hbm)


def _seg_sum_in(x_pad, src_pg, dst_pg, zeros_hbm):
    PTILE = PAGES // (2 * NTILES)

    @functools.partial(
        pl.kernel,
        mesh=_mesh,
        compiler_params=_sc_params,
        out_type=jax.ShapeDtypeStruct((2, NP, FW1), jnp.float32),
        scratch_types=[
            pltpu.VMEM((2, BLK, CH), jnp.int32),
            pltpu.VMEM((2, BLK, CH), jnp.int32),
            pltpu.VMEM((2, EBH, FW1), jnp.float32),
            pltpu.VMEM_SHARED((NP, FW1), jnp.float32),
            pltpu.SemaphoreType.DMA((3,)),
        ],
    )
    def k(tab_hbm, src_hbm, dst_hbm, z_hbm, out_hbm, src_v, dst_v, rows_v,
          acc, sem):
        c = lax.axis_index("c")
        s = lax.axis_index("s")
        r0 = s * ROWS_PER_TILE

        pltpu.sync_copy(z_hbm, acc.at[pl.ds(r0, ROWS_PER_TILE)])
        plsc.subcore_barrier()

        p0 = c * (PAGES // 2) + s * PTILE
        _edge_loop(tab_hbm, src_hbm, dst_hbm, src_v, dst_v, rows_v,
                   acc, sem, p0, PTILE)
        plsc.subcore_barrier()

        pltpu.sync_copy(acc.at[pl.ds(r0, ROWS_PER_TILE)],
                        out_hbm.at[c, pl.ds(r0, ROWS_PER_TILE)])

    return k(x_pad, src_pg, dst_pg, zeros_hbm)


GP = 25
EBLK = E // GP
PADB = (E2 - E) // GP


def _tc_prep(edge_index):
    def body(e_ref, src_ref, dst_ref):
        pid = pl.program_id(0)
        e = e_ref[...]
        srcr = e[0:1, :].reshape(EBLK // 128, 128)
        dstr = e[1:2, :].reshape(EBLK // 128, 128)
        li = (lax.broadcasted_iota(jnp.int32, (PADB // 128, 128), 0) * 128
              + lax.broadcasted_iota(jnp.int32, (PADB // 128, 128), 1)
              + pid * PADB)
        pad_src = (li * 13) % N
        pad_dst = N + li % (NP - N)
        src_ref[...] = jnp.concatenate([srcr, pad_src], axis=0)
        dst_ref[...] = jnp.concatenate([dstr, pad_dst], axis=0)

    RB = (EBLK + PADB) // 128
    return pl.pallas_call(
        body,
        grid=(GP,),
        in_specs=[
            pl.BlockSpec((2, EBLK), lambda i: (0, i)),
        ],
        out_specs=[
            pl.BlockSpec((RB, 128), lambda i: (i, 0)),
            pl.BlockSpec((RB, 128), lambda i: (i, 0)),
        ],
        out_shape=[
            jax.ShapeDtypeStruct((IDXR, 128), jnp.int32),
            jax.ShapeDtypeStruct((IDXR, 128), jnp.int32),
        ],
    )(edge_index)


R = 5120
RS = R // 16
RH = R // 8
RI = 10240
RIS = RI // 16
RIH = RI // 8

import numpy as _np
_Pnp = _np.zeros((8, 16, 128), _np.float32)
for _q in range(8):
    for _k in range(16):
        _Pnp[_q, _k, _q * 16 + _k] = 1.0


def _tc_in(S1p, xp, DA_La, DA_Lb, DA_Ra, DA_Rb, B0L, B0R, Wc, E8, Ea, Eb, P):
    def body(sp_ref, x_ref, dla_ref, dlb_ref, dra_ref, drb_ref, b0l_ref,
             b0r_ref, wc_ref, e8_ref, ea_ref, eb_ref, p_ref,
             hl_ref, hr_ref, inv_ref):
        f32 = jnp.float32
        Sp = sp_ref[0] + sp_ref[1]
        cnt = jnp.dot(Sp, wc_ref[...], preferred_element_type=f32)
        inv16 = 1.0 / jnp.maximum(cnt, 1.0)
        invp8 = jnp.dot(inv16, e8_ref[...], preferred_element_type=f32)
        aggp = Sp * invp8
        xb = x_ref[...]
        dn = (((0,), (0,)), ((), ()))

        def half(da, db, b0h):
            ha = jnp.dot(aggp, da, preferred_element_type=f32)
            hb = jnp.dot(aggp, db, preferred_element_type=f32)
            hp = jnp.stack([ha, hb], axis=1).reshape(RIH, 128)
            hx = lax.dot_general(xb, b0h, dn,
                                 preferred_element_type=f32)
            hxr = hx.reshape(RIH, 8, 16)
            for q in range(8):
                hp = hp + jnp.dot(hxr[:, q, :], p_ref[q],
                                  preferred_element_type=f32)
            return hp

        hl_ref[...] = half(dla_ref[...], dlb_ref[...], b0l_ref[...])
        hr_ref[...] = half(dra_ref[...], drb_ref[...], b0r_ref[...])
        ia = jnp.dot(inv16, ea_ref[...], preferred_element_type=f32)
        ib = jnp.dot(inv16, eb_ref[...], preferred_element_type=f32)
        inv_ref[...] = jnp.stack([ia, ib], axis=1).reshape(RIH, 128)

    full = lambda shape: pl.BlockSpec(shape, lambda i: tuple(0 for _ in shape))
    return pl.pallas_call(
        body,
        grid=(NP // RI,),
        in_specs=[
            pl.BlockSpec((2, RIS, 128), lambda i: (0, i, 0)),
            pl.BlockSpec((4, RI), lambda i: (0, i)),
            full((128, 128)), full((128, 128)), full((128, 128)),
            full((128, 128)), full((4, 16)), full((4, 16)),
            full((128, 16)), full((16, 128)), full((16, 128)),
            full((16, 128)), full((8, 16, 128)),
        ],
        out_specs=[
            pl.BlockSpec((RIH, 128), lambda i: (i, 0)),
            pl.BlockSpec((RIH, 128), lambda i: (i, 0)),
            pl.BlockSpec((RIH, 128), lambda i: (i, 0)),
        ],
        out_shape=[
            jax.ShapeDtypeStruct((NP * HALF // 128, 128), jnp.float32),
            jax.ShapeDtypeStruct((NP * HALF // 128, 128), jnp.float32),
            jax.ShapeDtypeStruct((NP * HALF // 128, 128), jnp.float32),
        ],
    )(S1p, xp, DA_La, DA_Lb, DA_Ra, DA_Rb, B0L, B0R, Wc, E8, Ea, Eb, P)


def _tc_mid(S, hl, hr, invp, DAL, DAR, DBL, DBR, DAL2, DAR2, DBL2, DBR2):
    def body(s_ref, hl_ref, hr_ref, inv_ref, a1_ref, a2_ref, b1_ref, b2_ref,
             a3_ref, a4_ref, b3_ref, b4_ref, ol_ref, or_ref):
        f32 = jnp.float32
        iv = inv_ref[...]
        aggL = s_ref[0] * iv
        aggR = s_ref[1] * iv
        hL = hl_ref[...]
        hR = hr_ref[...]

        def z(da, db, ba, bb):
            return (jnp.dot(aggL, da, preferred_element_type=f32)
                    + jnp.dot(aggR, db, preferred_element_type=f32)
                    + jnp.dot(hL, ba, preferred_element_type=f32)
                    + jnp.dot(hR, bb, preferred_element_type=f32))

        ol_ref[...] = jax.nn.sigmoid(
            z(a1_ref[...], a2_ref[...], b1_ref[...], b2_ref[...]))
        or_ref[...] = jax.nn.sigmoid(
            z(a3_ref[...], a4_ref[...], b3_ref[...], b4_ref[...]))

    full = lambda: pl.BlockSpec((128, 128), lambda i: (0, 0))
    blk = lambda: pl.BlockSpec((RH, 128), lambda i: (i, 0))
    return pl.pallas_call(
        body,
        grid=(NP // R,),
        in_specs=[pl.BlockSpec((2, RH, 128), lambda i: (0, i, 0)),
                  blk(), blk(), blk(),
                  full(), full(), full(), full(),
                  full(), full(), full(), full()],
        out_specs=[blk(), blk()],
        out_shape=[
            jax.ShapeDtypeStruct((NP * HALF // 128, 128), jnp.float32),
            jax.ShapeDtypeStruct((NP * HALF // 128, 128), jnp.float32),
        ],
    )(S, hl, hr, invp, DAL, DAR, DBL, DBR, DAL2, DAR2, DBL2, DBR2)


def _tc_out(S, hl, hr, invp, DAL, DAR, DBL, DBR, DAL2, DAR2, DBL2, DBR2,
            DWl, DWr, bias):
    def body(s_ref, hl_ref, hr_ref, inv_ref, a1_ref, a2_ref, b1_ref, b2_ref,
             a3_ref, a4_ref, b3_ref, b4_ref, wl_ref, wr_ref, bias_ref,
             o_ref):
        f32 = jnp.float32
        iv = inv_ref[...]
        aggL = s_ref[0] * iv
        aggR = s_ref[1] * iv
        hL = hl_ref[...]
        hR = hr_ref[...]

        def z(da, db, ba, bb):
            return (jnp.dot(aggL, da, preferred_element_type=f32)
                    + jnp.dot(aggR, db, preferred_element_type=f32)
                    + jnp.dot(hL, ba, preferred_element_type=f32)
                    + jnp.dot(hR, bb, preferred_element_type=f32))

        oL = jax.nn.sigmoid(
            z(a1_ref[...], a2_ref[...], b1_ref[...], b2_ref[...]))
        oR = jax.nn.sigmoid(
            z(a3_ref[...], a4_ref[...], b3_ref[...], b4_ref[...]))
        dn = (((0,), (1,)), ((), ()))
        tL = lax.dot_general(wl_ref[...], oL, dn, preferred_element_type=f32)
        tR = lax.dot_general(wr_ref[...], oR, dn, preferred_element_type=f32)
        o_ref[...] = tL + tR + bias_ref[0, 0]

    full = lambda: pl.BlockSpec((128, 128), lambda i: (0, 0))
    blk = lambda: pl.BlockSpec((RH, 128), lambda i: (i, 0))
    return pl.pallas_call(
        body,
        grid=(NP // R,),
        in_specs=[pl.BlockSpec((2, RH, 128), lambda i: (0, i, 0)),
                  blk(), blk(), blk(),
                  full(), full(), full(), full(),
                  full(), full(), full(), full(),
                  pl.BlockSpec((128, 8), lambda i: (0, 0)),
                  pl.BlockSpec((128, 8), lambda i: (0, 0)),
                  pl.BlockSpec((1, 1), lambda i: (0, 0))],
        out_specs=pl.BlockSpec((8, RH), lambda i: (0, i)),
        out_shape=jax.ShapeDtypeStruct((8, NP // 8), jnp.float32),
    )(S, hl, hr, invp, DAL, DAR, DBL, DBR, DAL2, DAR2, DBL2, DBR2,
      DWl, DWr, bias)


def kernel(x, edge_index, A0, B0, A1, B1, A2, B2, W_out, b_out):
    f32 = jnp.float32
    ei = edge_index.astype(jnp.int32)
    src_pg, dst_pg = _tc_prep(ei)
    x_pad = jnp.concatenate(
        [x, jnp.ones((N, 1), f32), jnp.zeros((N, 3), f32)],
        axis=1)
    xpT = jnp.pad(x.T, ((0, 0), (0, NP - N)))
    z16 = jnp.zeros((ROWS_PER_TILE, HALF), f32)
    z8 = jnp.zeros((ROWS_PER_TILE, FW1), f32)

    I8 = jnp.eye(8, dtype=f32)
    I16 = jnp.eye(16, dtype=f32)
    A0p = jnp.pad(A0, ((0, 4), (0, 0)))
    M_L = jnp.kron(I16, A0p[:, :HALF])
    M_R = jnp.kron(I16, A0p[:, HALF:])
    E8 = jnp.kron(I16, jnp.ones((1, 8), f32))
    EE = jnp.kron(I16, jnp.ones((1, 16), f32))
    onehot4 = jnp.zeros((8, 1), f32).at[4, 0].set(1.0)
    Wc = jnp.kron(I16, onehot4)
    P = jnp.asarray(_Pnp)

    def dmats(A, B):
        return (jnp.kron(I8, A[:HALF, :HALF]), jnp.kron(I8, A[HALF:, :HALF]),
                jnp.kron(I8, B[:HALF, :HALF]), jnp.kron(I8, B[HALF:, :HALF]),
                jnp.kron(I8, A[:HALF, HALF:]), jnp.kron(I8, A[HALF:, HALF:]),
                jnp.kron(I8, B[:HALF, HALF:]), jnp.kron(I8, B[HALF:, HALF:]))

    DWl = jnp.kron(I8, W_out[:HALF, :])
    DWr = jnp.kron(I8, W_out[HALF:, :])

    S1p = _seg_sum_in(x_pad, src_pg, dst_pg, z8)
    S1v = S1p.reshape(2, NP * FW1 // 128, 128)
    h0l, h0r, invp = _tc_in(S1v, xpT, M_L[:, :128], M_L[:, 128:],
                            M_R[:, :128], M_R[:, 128:], B0[:, :HALF],
                            B0[:, HALF:], Wc, E8, EE[:, :128], EE[:, 128:], P)
    S2 = _seg_sum_wide(h0l.reshape(NP, HALF), h0r.reshape(NP, HALF),
                       src_pg, dst_pg, z16)
    S2v = S2.reshape(2, NP * HALF // 128, 128)
    h1l, h1r = _tc_mid(S2v, h0l, h0r, invp, *dmats(A1, B1))
    S3 = _seg_sum_wide(h1l.reshape(NP, HALF), h1r.reshape(NP, HALF),
                       src_pg, dst_pg, z16)
    S3v = S3.reshape(2, NP * HALF // 128, 128)
    out = _tc_out(S3v, h1l, h1r, invp, *dmats(A2, B2), DWl, DWr,
                  b_out.reshape(1, 1))
    return out.T.reshape(NP)[:N]
